# Initial kernel scaffold; baseline (speedup 1.0000x reference)
import jax
import jax.numpy as jnp
from jax.experimental import pallas as pl


def kernel(x, edge_index, W_t, W1_0, b1_0, W2_0, b2_0, g0, bt0, W1s, b1s, W2s, b2s, gs, bs):
    raise NotImplementedError("write your pallas kernel here")


# speedup vs baseline: 2.6860x; 2.6860x over previous
import functools

import numpy as np

import jax
import jax.numpy as jnp
from jax import lax
from jax.experimental import pallas as pl
from jax.experimental.pallas import tpu as pltpu
from jax.experimental.pallas import tpu_sc as plsc

N = 10000
E = 320000
DIN = 128
DH = 64

NC = 2
NS = 16
NW = NC * NS

K = 128
SUB = 16
CH = 80
SHARE = CH * K
NPAD = 10240
ROWS_PER = NPAD // NS


def _tile_bounds():
  half = E // NC
  units, rem = divmod(half // 320, NS)
  sizes = [(units + 1) * 320 if i < rem else units * 320 for i in range(NS)]
  b = [0]
  for c in range(NC):
    for s in sizes:
      b.append(b[-1] + s)
  return b


_BOUNDS = _tile_bounds()
_TILE_OF_POS = np.searchsorted(np.asarray(_BOUNDS[1:]), np.arange(E),
                               side='right').astype(np.int32)
_POS_RAW = np.stack([
    np.arange(_BOUNDS[t], _BOUNDS[t] + SHARE, dtype=np.int64)
    for t in range(NW)]).reshape(NW, CH, K)
_PAD_MASK = np.stack([
    (np.arange(SHARE) >= (_BOUNDS[t + 1] - _BOUNDS[t]))
    for t in range(NW)]).reshape(NW, CH, K)
_POS_CLIP = np.minimum(_POS_RAW, E - 1).astype(np.int32)
_TRASH = (N + 64 + np.arange(NW, dtype=np.int32))[:, None, None] + np.zeros(
    (NW, CH, K), np.int32)


def _seg_body(h_hbm, src_hbm, dst_hbm, zeros_hbm, out_hbm,
              src_v, dst_v, rows_v, agg_sh, sem):
  c = lax.axis_index("c")
  s = lax.axis_index("s")
  wid = c * NS + s
  pltpu.sync_copy(zeros_hbm.at[pl.ds(s * ROWS_PER, ROWS_PER)],
                  agg_sh.at[pl.ds(s * ROWS_PER, ROWS_PER)])
  pltpu.sync_copy(src_hbm.at[wid], src_v)
  pltpu.sync_copy(dst_hbm.at[wid], dst_v)
  plsc.subcore_barrier()

  def chunk(j, carry):
    pltpu.async_copy(h_hbm.at[src_v.at[j]], rows_v, sem).wait()
    for b in range(K // SUB):
      pltpu.sync_copy(rows_v.at[pl.ds(b * SUB, SUB)],
                      agg_sh.at[dst_v.at[j, pl.ds(b * SUB, SUB)]], add=True)
    return carry

  lax.fori_loop(0, CH, chunk, 0)
  plsc.subcore_barrier()
  pltpu.sync_copy(agg_sh.at[pl.ds(s * ROWS_PER, ROWS_PER)],
                  out_hbm.at[c, pl.ds(s * ROWS_PER, ROWS_PER)])


def _make_seg_sum(d):
  mesh = plsc.VectorSubcoreMesh(core_axis_name="c", subcore_axis_name="s")
  return pl.kernel(
      _seg_body,
      out_type=jax.ShapeDtypeStruct((NC, NPAD, d), jnp.float32),
      mesh=mesh,
      scratch_types=[
          pltpu.VMEM((CH, K), jnp.int32),
          pltpu.VMEM((CH, K), jnp.int32),
          pltpu.VMEM((K, d), jnp.float32),
          pltpu.VMEM_SHARED((NPAD, d), jnp.float32),
          pltpu.SemaphoreType.DMA,
      ],
      compiler_params=pltpu.CompilerParams(use_tc_tiling_on_sc=False),
  )


def _tc_first(x_ref, wt_ref, out_ref):
  out_ref[...] = jnp.dot(x_ref[...], wt_ref[...].T)


def _tc_mlp(h_ref, parts_ref, dtile_ref, w1_ref, b1_ref, w2_ref, b2_ref,
            out_ref):
  agg = parts_ref[0, :N, :] + parts_ref[1, :N, :]
  rows = lax.broadcasted_iota(jnp.int32, (N, 1), 0)
  for t in range(1, NW):
    d = dtile_ref[t]
    dummy = parts_ref[t // NS, N + t, :]
    agg = agg + jnp.where(rows == d, dummy[None, :], 0.0)
  z = h_ref[...] + agg
  t1 = jnp.maximum(jnp.dot(z, w1_ref[...].T) + b1_ref[...], 0.0)
  out_ref[...] = jnp.dot(t1, w2_ref[...].T) + b2_ref[...]


def _tc_bn(t_ref, mu_ref, var_ref, g_ref, bt_ref, out_ref):
  h = (g_ref[...] * (t_ref[...] - mu_ref[...]) /
       jnp.sqrt(var_ref[...] + 1e-5) + bt_ref[...])
  out_ref[...] = jnp.maximum(h, 0.0)


def _tc_mlp_call(h, parts, dtile, w1, b1, w2, b2):
  return pl.pallas_call(
      _tc_mlp,
      out_shape=jax.ShapeDtypeStruct((N, DH), jnp.float32),
      in_specs=[
          pl.BlockSpec(memory_space=pltpu.VMEM),
          pl.BlockSpec(memory_space=pltpu.VMEM),
          pl.BlockSpec(memory_space=pltpu.SMEM),
          pl.BlockSpec(memory_space=pltpu.VMEM),
          pl.BlockSpec(memory_space=pltpu.VMEM),
          pl.BlockSpec(memory_space=pltpu.VMEM),
          pl.BlockSpec(memory_space=pltpu.VMEM),
      ],
  )(h, parts, dtile, w1, b1, w2, b2)


def _tc_bn_call(t, mu, var, g, bt):
  return pl.pallas_call(
      _tc_bn,
      out_shape=jax.ShapeDtypeStruct((N, DH), jnp.float32),
  )(t, mu, var, g, bt)


def kernel(x, edge_index, W_t, W1_0, b1_0, W2_0, b2_0, g0, bt0,
           W1s, b1s, W2s, b2s, gs, bs):
  src = edge_index[0]
  dst = edge_index[1]
  order = jnp.argsort(dst, stable=True)
  src_s = src[order]
  dst_s = dst[order]
  tile_of_pos = jnp.asarray(_TILE_OF_POS)
  bounds = jnp.asarray(np.asarray(_BOUNDS[:NW], np.int32))
  d_head = dst_s[bounds]
  head_of_pos = d_head[tile_of_pos]
  dst_r = jnp.where((tile_of_pos > 0) & (dst_s == head_of_pos),
                    N + tile_of_pos, dst_s)
  pos = jnp.asarray(_POS_CLIP)
  padm = jnp.asarray(_PAD_MASK)
  srcp = jnp.where(padm, 0, src_s[pos])
  dstp = jnp.where(padm, jnp.asarray(_TRASH), dst_r[pos])
  zeros128 = jnp.zeros((NPAD, DIN), jnp.float32)
  zeros64 = jnp.zeros((NPAD, DH), jnp.float32)
  dtile = d_head.astype(jnp.int32)

  seg128 = _make_seg_sum(DIN)
  seg64 = _make_seg_sum(DH)

  h = pl.pallas_call(
      _tc_first,
      out_shape=jax.ShapeDtypeStruct((N, DIN), jnp.float32),
  )(x, W_t)

  layer_ws = [
      (W1_0, b1_0, W2_0, b2_0, g0, bt0),
      (W1s[0], b1s[0], W2s[0], b2s[0], gs[0], bs[0]),
      (W1s[1], b1s[1], W2s[1], b2s[1], gs[1], bs[1]),
      (W1s[2], b1s[2], W2s[2], b2s[2], gs[2], bs[2]),
  ]
  for l in range(4):
    if l == 0:
      parts = seg128(h, srcp, dstp, zeros128)
    else:
      parts = seg64(h, srcp, dstp, zeros64)
    w1, b1, w2, b2, g, bt = layer_ws[l]
    t = _tc_mlp_call(h, parts, dtile, w1, b1, w2, b2)
    mu = t.mean(axis=0, keepdims=True)
    var = t.var(axis=0, keepdims=True)
    h = _tc_bn_call(t, mu, var, g, bt)
  return h
